# baseline (speedup 1.0000x reference)
import functools

import jax
import jax.numpy as jnp
from jax.experimental import pallas as pl
from jax.experimental.pallas import tpu as pltpu

_N_ROWS = 10000
_N_COLS = 128
_N_BLOCKS = 10
_BLOCK_ROWS = _N_ROWS // _N_BLOCKS


def _body(x_hbm, h_hbm, u_ref, t_ref, buf, in_sems, out_sems):
    def in_copy(i, slot):
        return pltpu.make_async_copy(
            x_hbm.at[pl.ds(i * _BLOCK_ROWS, _BLOCK_ROWS), :],
            buf.at[slot],
            in_sems.at[slot],
        )

    def out_copy(i, slot):
        return pltpu.make_async_copy(
            buf.at[slot],
            h_hbm.at[pl.ds(i * _BLOCK_ROWS, _BLOCK_ROWS), :],
            out_sems.at[slot],
        )

    in_copy(0, 0).start()

    def loop(i, acc):
        slot = jax.lax.rem(i, 2)
        nxt = jax.lax.rem(i + 1, 2)

        @pl.when(i + 1 < _N_BLOCKS)
        def _():
            @pl.when(i >= 1)
            def _():
                out_copy(i - 1, nxt).wait()

            in_copy(i + 1, nxt).start()

        in_copy(i, slot).wait()
        out_copy(i, slot).start()
        return acc + jnp.sum(buf[slot], axis=0, keepdims=True)

    acc = jax.lax.fori_loop(
        0, _N_BLOCKS, loop, jnp.zeros((1, _N_COLS), jnp.float32)
    )

    out_copy(_N_BLOCKS - 2, (_N_BLOCKS - 2) % 2).wait()
    out_copy(_N_BLOCKS - 1, (_N_BLOCKS - 1) % 2).wait()

    u = acc * (1.0 / _N_ROWS)
    u_ref[...] = u
    m = jnp.max(u, axis=1, keepdims=True)
    e = jnp.exp(u - m)
    t_ref[...] = e / jnp.sum(e, axis=1, keepdims=True)


@functools.partial(jax.jit, static_argnames=())
def _fused(x):
    h, u, u_top = pl.pallas_call(
        _body,
        in_specs=[pl.BlockSpec(memory_space=pltpu.MemorySpace.HBM)],
        out_specs=[
            pl.BlockSpec(memory_space=pltpu.MemorySpace.HBM),
            pl.BlockSpec(memory_space=pltpu.MemorySpace.VMEM),
            pl.BlockSpec(memory_space=pltpu.MemorySpace.VMEM),
        ],
        out_shape=[
            jax.ShapeDtypeStruct((_N_ROWS, _N_COLS), jnp.float32),
            jax.ShapeDtypeStruct((1, _N_COLS), jnp.float32),
            jax.ShapeDtypeStruct((1, _N_COLS), jnp.float32),
        ],
        scratch_shapes=[
            pltpu.VMEM((2, _BLOCK_ROWS, _N_COLS), jnp.float32),
            pltpu.SemaphoreType.DMA((2,)),
            pltpu.SemaphoreType.DMA((2,)),
        ],
    )(x)
    return h, u, u_top


def kernel(x, edge_index):
    del edge_index
    return _fused(x)
